# baseline (speedup 1.0000x reference)
import functools

import jax
import jax.numpy as jnp
from jax import lax
from jax.experimental import pallas as pl
from jax.experimental.pallas import tpu as pltpu
from jax.experimental.pallas import tpu_sc as plsc

_BATCH = 16384
_FIELDS = 26
_DIM = 64
_B = _BATCH * _FIELDS

_NC = 2
_NS = 16
_NW = _NC * _NS
_BPW = _B // _NW
_CHUNK = 256
_NCHUNK = _BPW // _CHUNK
_K = 4

_mesh = plsc.VectorSubcoreMesh(core_axis_name="c", subcore_axis_name="s")


@functools.partial(
    pl.kernel,
    out_type=jax.ShapeDtypeStruct((_B, _DIM), jnp.float32),
    mesh=_mesh,
    scratch_types=[
        pltpu.VMEM((_NCHUNK, _CHUNK), jnp.int32),
        [pltpu.VMEM((_CHUNK, _DIM), jnp.float32) for _ in range(_K)],
        [pltpu.SemaphoreType.DMA for _ in range(_K)],
    ],
    compiler_params=pltpu.CompilerParams(use_tc_tiling_on_sc=False),
)
def _sc_gather(table_hbm, idx_hbm, out_hbm, idx_v, bufs, sems):
    wid = lax.axis_index("s") * _NC + lax.axis_index("c")
    base = wid * _BPW
    pltpu.sync_copy(idx_hbm.at[wid], idx_v)
    for b in range(_K):
        pltpu.async_copy(table_hbm.at[idx_v.at[b]], bufs[b], sems[b])

    @pl.loop(0, _NCHUNK, step=_K)
    def _group(g):
        for b in range(_K):
            i = g + b
            pltpu.make_async_copy(table_hbm.at[idx_v.at[i]], bufs[b], sems[b]).wait()
            pltpu.sync_copy(bufs[b], out_hbm.at[pl.ds(base + i * _CHUNK, _CHUNK)])

            @pl.when(i + _K < _NCHUNK)
            def _refill():
                pltpu.async_copy(table_hbm.at[idx_v.at[i + _K]], bufs[b], sems[b])


_ROWS_BLK = 256


def _retile_body(x_ref, o_ref):
    for f in range(_FIELDS):
        o_ref[:, f, :] = x_ref[:, f * _DIM:(f + 1) * _DIM]


def _tc_retile(flat):
    return pl.pallas_call(
        _retile_body,
        grid=(_BATCH // _ROWS_BLK,),
        in_specs=[pl.BlockSpec((_ROWS_BLK, _FIELDS * _DIM), lambda i: (i, 0))],
        out_specs=pl.BlockSpec((_ROWS_BLK, _FIELDS, _DIM), lambda i: (i, 0, 0)),
        out_shape=jax.ShapeDtypeStruct((_BATCH, _FIELDS, _DIM), jnp.float32),
    )(flat)


def kernel(x, weight):
    idx = jnp.maximum(x.reshape(_NW, _NCHUNK, _CHUNK).astype(jnp.int32), 0)
    g = _sc_gather(weight, idx)
    return _tc_retile(g.reshape(_BATCH, _FIELDS * _DIM))


# speedup vs baseline: 1.3909x; 1.3909x over previous
import functools

import jax
import jax.numpy as jnp
from jax import lax
from jax.experimental import pallas as pl
from jax.experimental.pallas import tpu as pltpu
from jax.experimental.pallas import tpu_sc as plsc

_BATCH = 16384
_FIELDS = 26
_DIM = 64

_NC = 2
_NS = 16
_NW = _NC * _NS
_ROWS_W = _BATCH // _NW
_CB = 16
_CHUNK = _CB * _FIELDS
_NCHUNK = _ROWS_W // _CB
_K = 4

_mesh = plsc.VectorSubcoreMesh(core_axis_name="c", subcore_axis_name="s")


@functools.partial(
    pl.kernel,
    out_type=jax.ShapeDtypeStruct((_BATCH, _FIELDS, _DIM), jnp.float32),
    mesh=_mesh,
    scratch_types=[
        pltpu.VMEM((_NCHUNK, _CHUNK), jnp.int32),
        [pltpu.VMEM((_CHUNK, _DIM), jnp.float32) for _ in range(_K)],
        [pltpu.SemaphoreType.DMA for _ in range(_K)],
    ],
    compiler_params=pltpu.CompilerParams(use_tc_tiling_on_sc=False),
)
def _sc_gather(table_hbm, idx_hbm, out_hbm, idx_v, bufs, sems):
    wid = lax.axis_index("s") * _NC + lax.axis_index("c")
    row0 = wid * _ROWS_W
    pltpu.sync_copy(idx_hbm.at[wid], idx_v)
    for b in range(_K):
        pltpu.async_copy(table_hbm.at[idx_v.at[b]], bufs[b], sems[b])

    @pl.loop(0, _NCHUNK, step=_K)
    def _group(g):
        for b in range(_K):
            i = g + b
            pltpu.make_async_copy(table_hbm.at[idx_v.at[i]], bufs[b], sems[b]).wait()
            for k in range(_CB):
                pltpu.sync_copy(bufs[b].at[pl.ds(k * _FIELDS, _FIELDS)],
                                out_hbm.at[row0 + i * _CB + k])

            @pl.when(i + _K < _NCHUNK)
            def _refill():
                pltpu.async_copy(table_hbm.at[idx_v.at[i + _K]], bufs[b], sems[b])


def kernel(x, weight):
    idx = jnp.maximum(x.reshape(_NW, _NCHUNK, _CHUNK).astype(jnp.int32), 0)
    return _sc_gather(weight, idx)


# speedup vs baseline: 1.3930x; 1.0015x over previous
import functools

import jax
import jax.numpy as jnp
from jax import lax
from jax.experimental import pallas as pl
from jax.experimental.pallas import tpu as pltpu
from jax.experimental.pallas import tpu_sc as plsc

_BATCH = 16384
_FIELDS = 26
_DIM = 64
_B = _BATCH * _FIELDS

_NC = 2
_NS = 16
_NW = _NC * _NS
_BPW = _B // _NW
_CHUNK = 416
_NCHUNK = _BPW // _CHUNK
_K = 4

_mesh = plsc.VectorSubcoreMesh(core_axis_name="c", subcore_axis_name="s")


@functools.partial(
    pl.kernel,
    out_type=jax.ShapeDtypeStruct((_B, _DIM), jnp.float32),
    mesh=_mesh,
    scratch_types=[
        pltpu.VMEM((_BPW,), jnp.int32),
        [pltpu.VMEM((_CHUNK, _DIM), jnp.float32) for _ in range(_K)],
        [pltpu.SemaphoreType.DMA for _ in range(_K)],
    ],
    compiler_params=pltpu.CompilerParams(use_tc_tiling_on_sc=False),
)
def _sc_gather(table_hbm, idx_hbm, out_hbm, idx_v, bufs, sems):
    wid = lax.axis_index("s") * _NC + lax.axis_index("c")
    base = wid * _BPW
    pltpu.sync_copy(idx_hbm.at[pl.ds(base, _BPW)], idx_v)
    for b in range(_K):
        pltpu.async_copy(table_hbm.at[idx_v.at[pl.ds(b * _CHUNK, _CHUNK)]],
                         bufs[b], sems[b])

    @pl.loop(0, _NCHUNK, step=_K)
    def _group(g):
        for b in range(_K):
            i = g + b
            pltpu.make_async_copy(
                table_hbm.at[idx_v.at[pl.ds(i * _CHUNK, _CHUNK)]],
                bufs[b], sems[b]).wait()
            pltpu.sync_copy(
                bufs[b],
                out_hbm.at[pl.ds(base + i * _CHUNK, _CHUNK)])

            @pl.when(i + _K < _NCHUNK)
            def _refill():
                pltpu.async_copy(
                    table_hbm.at[idx_v.at[pl.ds((i + _K) * _CHUNK, _CHUNK)]],
                    bufs[b], sems[b])


def kernel(x, weight):
    idx = jnp.maximum(x.reshape(_B).astype(jnp.int32), 0)
    out = _sc_gather(weight, idx)
    return out.reshape(_BATCH, _FIELDS, _DIM)
